# baseline (speedup 1.0000x reference)
import functools

import jax
import jax.numpy as jnp
from jax import lax
from jax.experimental import pallas as pl
from jax.experimental.pallas import tpu as pltpu
from jax.experimental.pallas import tpu_sc as plsc

HID = 64
N_NODES = 50000
N_EDGES = 800000
HALF = 25000
TILE_ROWS = 1568
ACC_ROWS = 16 * TILE_ROWS
E_PER_TILE = N_EDGES // 16
SUPER = 2000
CHUNK = 128
NSUP = E_PER_TILE // SUPER
CBUF = SUPER + CHUNK + 16


def _encode_body(x_ref, w1_ref, b1_ref, w2_ref, b2_ref, o_ref):
    h = jnp.dot(x_ref[...], w1_ref[...], preferred_element_type=jnp.float32)
    h = jnp.maximum(h + b1_ref[...], 0.0)
    h = jnp.dot(h, w2_ref[...], preferred_element_type=jnp.float32)
    o_ref[...] = jnp.maximum(h + b2_ref[...], 0.0)


def _encode(x, p1, p2):
    n, din = x.shape
    blk = 2000
    return pl.pallas_call(
        _encode_body,
        grid=(n // blk,),
        in_specs=[
            pl.BlockSpec((blk, din), lambda i: (i, 0)),
            pl.BlockSpec((din, HID), lambda i: (0, 0)),
            pl.BlockSpec((1, HID), lambda i: (0, 0)),
            pl.BlockSpec((HID, HID), lambda i: (0, 0)),
            pl.BlockSpec((1, HID), lambda i: (0, 0)),
        ],
        out_specs=pl.BlockSpec((blk, HID), lambda i: (i, 0)),
        out_shape=jax.ShapeDtypeStruct((n, HID), jnp.float32),
    )(x, p1["W"], p1["b"].reshape(1, HID), p2["W"], p2["b"].reshape(1, HID))


def _update_body(m_ref, h_ref, wm_ref, wh_ref, b_ref, o_ref):
    z = (jnp.dot(m_ref[...], wm_ref[...], preferred_element_type=jnp.float32)
         + jnp.dot(h_ref[...], wh_ref[...], preferred_element_type=jnp.float32)
         + b_ref[...])
    mu = jnp.mean(z, axis=-1, keepdims=True)
    zc = z - mu
    var = jnp.mean(zc * zc, axis=-1, keepdims=True)
    o_ref[...] = jnp.maximum(zc * lax.rsqrt(var + 1e-5), 0.0)


def _update(msg, h, wm, wh, b):
    n = h.shape[0]
    blk = 2000
    return pl.pallas_call(
        _update_body,
        grid=(n // blk,),
        in_specs=[
            pl.BlockSpec((blk, HID), lambda i: (i, 0)),
            pl.BlockSpec((blk, HID), lambda i: (i, 0)),
            pl.BlockSpec((HID, HID), lambda i: (0, 0)),
            pl.BlockSpec((HID, HID), lambda i: (0, 0)),
            pl.BlockSpec((1, HID), lambda i: (0, 0)),
        ],
        out_specs=pl.BlockSpec((blk, HID), lambda i: (i, 0)),
        out_shape=jax.ShapeDtypeStruct((n, HID), jnp.float32),
    )(msg, h, wm, wh, b.reshape(1, HID))


def _pred_body(h_ref, wp_ref, bp_ref, wo_ref, bo_ref, o_ref):
    h = jnp.dot(h_ref[...], wp_ref[...], preferred_element_type=jnp.float32)
    h = jnp.maximum(h + bp_ref[...], 0.0)
    o_ref[...] = jnp.sum(h * wo_ref[...], axis=1) + bo_ref[0, 0]


def _pred(h, pred_p, out_p):
    n = h.shape[0]
    blk = 2048
    grid = (n + blk - 1) // blk
    out = pl.pallas_call(
        _pred_body,
        grid=(grid,),
        in_specs=[
            pl.BlockSpec((blk, HID), lambda i: (i, 0)),
            pl.BlockSpec((HID, HID), lambda i: (0, 0)),
            pl.BlockSpec((1, HID), lambda i: (0, 0)),
            pl.BlockSpec((1, HID), lambda i: (0, 0)),
            pl.BlockSpec((1, 1), lambda i: (0, 0)),
        ],
        out_specs=pl.BlockSpec((blk,), lambda i: (i,)),
        out_shape=jax.ShapeDtypeStruct((grid * blk,), jnp.float32),
    )(h, pred_p["W"], pred_p["b"].reshape(1, HID),
      out_p["W"].reshape(1, HID), out_p["b"].reshape(1, 1))
    return out[:n]


NBUF = 2


def _segsum_body(table, gidx, sidx, zrows, out, acc, gsb, ssb, cg, cd, dbuf,
                 rows, gsem):
    c = lax.axis_index("c")
    s = lax.axis_index("s")
    tile_base = s * TILE_ROWS
    pltpu.sync_copy(zrows, acc.at[pl.ds(tile_base, TILE_ROWS)])
    plsc.subcore_barrier()

    half_base = c * HALF
    ebase = s * E_PER_TILE
    trash = jnp.full((16,), HALF, jnp.int32)
    zero16 = jnp.zeros((16,), jnp.int32)

    def sup_body(j, carry):
        sb = ebase + j * SUPER
        pltpu.sync_copy(gidx.at[pl.ds(sb, SUPER)], gsb)
        pltpu.sync_copy(sidx.at[pl.ds(sb, SUPER)], ssb)

        def cp_body(i, w):
            g = gsb[pl.ds(i * 16, 16)]
            d = ssb[pl.ds(i * 16, 16)]
            loc = d - half_base
            ok = (loc >= 0) & (loc < HALF)
            plsc.store_compressed(cg.at[pl.ds(w, 16)], g, mask=ok)
            plsc.store_compressed(cd.at[pl.ds(w, 16)], loc, mask=ok)
            return w + jnp.sum(ok.astype(jnp.int32))

        w = lax.fori_loop(0, SUPER // 16, cp_body, jnp.int32(0))
        for k in range(CHUNK // 16):
            cg[pl.ds(w + k * 16, 16)] = zero16
            cd[pl.ds(w + k * 16, 16)] = trash
        nch = (w + CHUNK - 1) // CHUNK

        for b in range(NBUF):
            @pl.when(b < nch)
            def _():
                pltpu.async_copy(table.at[cg.at[pl.ds(b * CHUNK, CHUNK)]],
                                 rows.at[b], gsem.at[b])

        def ch_body(qq, carry2):
            for b in range(NBUF):
                q = qq * NBUF + b

                @pl.when(q < nch)
                def _():
                    off = q * CHUNK
                    pltpu.make_async_copy(
                        table.at[cg.at[pl.ds(off, CHUNK)]],
                        rows.at[b], gsem.at[b]).wait()
                    for t in range(CHUNK // 16):
                        dbuf[pl.ds(t * 16, 16)] = cd[pl.ds(off + t * 16, 16)]
                    pltpu.sync_copy(rows.at[b], acc.at[dbuf], add=True)

                    @pl.when(q + NBUF < nch)
                    def _():
                        pltpu.async_copy(
                            table.at[cg.at[pl.ds(off + NBUF * CHUNK, CHUNK)]],
                            rows.at[b], gsem.at[b])
            return carry2

        lax.fori_loop(0, (nch + NBUF - 1) // NBUF, ch_body, 0)
        return carry

    lax.fori_loop(0, NSUP, sup_body, 0)
    plsc.subcore_barrier()
    pltpu.sync_copy(acc.at[pl.ds(tile_base, TILE_ROWS)],
                    out.at[pl.ds(c * ACC_ROWS + tile_base, TILE_ROWS)])


@functools.cache
def _segsum_call():
    return pl.kernel(
        _segsum_body,
        out_type=jax.ShapeDtypeStruct((2 * ACC_ROWS, HID), jnp.float32),
        mesh=plsc.VectorSubcoreMesh(core_axis_name="c", subcore_axis_name="s",
                                    num_cores=2, num_subcores=16),
        scratch_types=[
            pltpu.VMEM_SHARED((ACC_ROWS, HID), jnp.float32),
            pltpu.VMEM((SUPER,), jnp.int32),
            pltpu.VMEM((SUPER,), jnp.int32),
            pltpu.VMEM((CBUF,), jnp.int32),
            pltpu.VMEM((CBUF,), jnp.int32),
            pltpu.VMEM((CHUNK,), jnp.int32),
            pltpu.VMEM((NBUF, CHUNK, HID), jnp.float32),
            pltpu.SemaphoreType.DMA((NBUF,)),
        ],
        compiler_params=pltpu.CompilerParams(use_tc_tiling_on_sc=False,
                                             needs_layout_passes=False),
    )


def _segment_sum(table, g_idx, s_idx, zrows):
    out = _segsum_call()(table, g_idx, s_idx, zrows)
    return out.reshape(2, ACC_ROWS, HID)[:, :HALF].reshape(N_NODES, HID)


def kernel(x_vals, x_cons, edge_index, params):
    hv = _encode(x_vals, *params["enc_v"])
    hc = _encode(x_cons, *params["enc_c"])
    row = edge_index[0].astype(jnp.int32)
    col = edge_index[1].astype(jnp.int32)
    zrows = jnp.zeros((TILE_ROWS, HID), jnp.float32)
    for layer in params["convs"]:
        msg_c = _segment_sum(hv, col, row, zrows)
        hc = _update(msg_c, hc, layer["Wv2c"], layer["Wcs"], layer["bc"])
        msg_v = _segment_sum(hc, row, col, zrows)
        hv = _update(msg_v, hv, layer["Wc2v"], layer["Wvs"], layer["bv"])
    return _pred(hv, params["pred"][0], params["out"])


# speedup vs baseline: 1.7588x; 1.7588x over previous
import functools

import jax
import jax.numpy as jnp
from jax import lax
from jax.experimental import pallas as pl
from jax.experimental.pallas import tpu as pltpu
from jax.experimental.pallas import tpu_sc as plsc

HID = 64
FHALF = HID // 2
N_NODES = 50000
N_EDGES = 800000
TILE_ROWS = 3128
ACC_ROWS = 16 * TILE_ROWS
E_PER_TILE = N_EDGES // 16
SUPER = 2000
CHUNK = 80
NSUP = E_PER_TILE // SUPER
NCH = SUPER // CHUNK


def _encode_body(x_ref, w1_ref, b1_ref, w2_ref, b2_ref, o_ref):
    h = jnp.dot(x_ref[...], w1_ref[...], preferred_element_type=jnp.float32)
    h = jnp.maximum(h + b1_ref[...], 0.0)
    h = jnp.dot(h, w2_ref[...], preferred_element_type=jnp.float32)
    o_ref[...] = jnp.maximum(h + b2_ref[...], 0.0)


def _encode(x, p1, p2):
    n, din = x.shape
    blk = 2000
    return pl.pallas_call(
        _encode_body,
        grid=(n // blk,),
        in_specs=[
            pl.BlockSpec((blk, din), lambda i: (i, 0)),
            pl.BlockSpec((din, HID), lambda i: (0, 0)),
            pl.BlockSpec((1, HID), lambda i: (0, 0)),
            pl.BlockSpec((HID, HID), lambda i: (0, 0)),
            pl.BlockSpec((1, HID), lambda i: (0, 0)),
        ],
        out_specs=pl.BlockSpec((blk, HID), lambda i: (i, 0)),
        out_shape=jax.ShapeDtypeStruct((n, HID), jnp.float32),
    )(x, p1["W"], p1["b"].reshape(1, HID), p2["W"], p2["b"].reshape(1, HID))


def _update_body(m_ref, h_ref, wm_ref, wh_ref, b_ref, o_ref):
    z = (jnp.dot(m_ref[...], wm_ref[...], preferred_element_type=jnp.float32)
         + jnp.dot(h_ref[...], wh_ref[...], preferred_element_type=jnp.float32)
         + b_ref[...])
    mu = jnp.mean(z, axis=-1, keepdims=True)
    zc = z - mu
    var = jnp.mean(zc * zc, axis=-1, keepdims=True)
    o_ref[...] = jnp.maximum(zc * lax.rsqrt(var + 1e-5), 0.0)


def _update(msg, h, wm, wh, b):
    n = h.shape[0]
    blk = 2000
    return pl.pallas_call(
        _update_body,
        grid=(n // blk,),
        in_specs=[
            pl.BlockSpec((blk, HID), lambda i: (i, 0)),
            pl.BlockSpec((blk, HID), lambda i: (i, 0)),
            pl.BlockSpec((HID, HID), lambda i: (0, 0)),
            pl.BlockSpec((HID, HID), lambda i: (0, 0)),
            pl.BlockSpec((1, HID), lambda i: (0, 0)),
        ],
        out_specs=pl.BlockSpec((blk, HID), lambda i: (i, 0)),
        out_shape=jax.ShapeDtypeStruct((n, HID), jnp.float32),
    )(msg, h, wm, wh, b.reshape(1, HID))


def _pred_body(h_ref, wp_ref, bp_ref, wo_ref, bo_ref, o_ref):
    h = jnp.dot(h_ref[...], wp_ref[...], preferred_element_type=jnp.float32)
    h = jnp.maximum(h + bp_ref[...], 0.0)
    o_ref[...] = jnp.sum(h * wo_ref[...], axis=1) + bo_ref[0, 0]


def _pred(h, pred_p, out_p):
    n = h.shape[0]
    blk = 2048
    grid = (n + blk - 1) // blk
    out = pl.pallas_call(
        _pred_body,
        grid=(grid,),
        in_specs=[
            pl.BlockSpec((blk, HID), lambda i: (i, 0)),
            pl.BlockSpec((HID, HID), lambda i: (0, 0)),
            pl.BlockSpec((1, HID), lambda i: (0, 0)),
            pl.BlockSpec((1, HID), lambda i: (0, 0)),
            pl.BlockSpec((1, 1), lambda i: (0, 0)),
        ],
        out_specs=pl.BlockSpec((blk,), lambda i: (i,)),
        out_shape=jax.ShapeDtypeStruct((grid * blk,), jnp.float32),
    )(h, pred_p["W"], pred_p["b"].reshape(1, HID),
      out_p["W"].reshape(1, HID), out_p["b"].reshape(1, 1))
    return out[:n]


NBUF = 5


def _segsum_body(table2, gidx, sidx, zrows, out, acc, gsb, ssb, gbufs, dbuf,
                 rows, gsem):
    c = lax.axis_index("c")
    s = lax.axis_index("s")
    tile_base = s * TILE_ROWS
    pltpu.sync_copy(zrows, acc.at[pl.ds(tile_base, TILE_ROWS)])
    plsc.subcore_barrier()

    ebase = s * E_PER_TILE

    def sup_body(j, carry):
        sb = ebase + j * SUPER
        pltpu.sync_copy(gidx.at[pl.ds(sb, SUPER)], gsb)
        pltpu.sync_copy(sidx.at[pl.ds(sb, SUPER)], ssb)

        def fill_gbuf(q, b):
            off = q * CHUNK
            for t in range(CHUNK // 16):
                g = gsb[pl.ds(off + t * 16, 16)]
                gbufs[b, pl.ds(t * 16, 16)] = g + g + c

        for b in range(NBUF):
            fill_gbuf(b, b)
            pltpu.async_copy(table2.at[gbufs.at[b]], rows.at[b], gsem.at[b])

        def ch_body(qq, carry2):
            for b in range(NBUF):
                q = qq * NBUF + b
                off = q * CHUNK
                pltpu.make_async_copy(table2.at[gbufs.at[b]],
                                      rows.at[b], gsem.at[b]).wait()
                for t in range(CHUNK // 16):
                    dbuf[pl.ds(t * 16, 16)] = ssb[pl.ds(off + t * 16, 16)]
                pltpu.sync_copy(rows.at[b], acc.at[dbuf], add=True)

                @pl.when(qq < NCH // NBUF - 1)
                def _():
                    fill_gbuf(q + NBUF, b)
                    pltpu.async_copy(table2.at[gbufs.at[b]],
                                     rows.at[b], gsem.at[b])
            return carry2

        return lax.fori_loop(0, NCH // NBUF, ch_body, carry)

    lax.fori_loop(0, NSUP, sup_body, 0)
    plsc.subcore_barrier()
    pltpu.sync_copy(acc.at[pl.ds(tile_base, TILE_ROWS)],
                    out.at[pl.ds(tile_base, TILE_ROWS), c])


@functools.cache
def _segsum_call():
    return pl.kernel(
        _segsum_body,
        out_type=jax.ShapeDtypeStruct((ACC_ROWS, 2, FHALF), jnp.float32),
        mesh=plsc.VectorSubcoreMesh(core_axis_name="c", subcore_axis_name="s",
                                    num_cores=2, num_subcores=16),
        scratch_types=[
            pltpu.VMEM_SHARED((ACC_ROWS, FHALF), jnp.float32),
            pltpu.VMEM((SUPER,), jnp.int32),
            pltpu.VMEM((SUPER,), jnp.int32),
            pltpu.VMEM((NBUF, CHUNK), jnp.int32),
            pltpu.VMEM((CHUNK,), jnp.int32),
            pltpu.VMEM((NBUF, CHUNK, FHALF), jnp.float32),
            pltpu.SemaphoreType.DMA((NBUF,)),
        ],
        compiler_params=pltpu.CompilerParams(use_tc_tiling_on_sc=False),
    )


def _segment_sum(table, g_idx, s_idx, zrows):
    table2 = table.reshape(2 * N_NODES, FHALF)
    out = _segsum_call()(table2, g_idx, s_idx, zrows)
    return out.reshape(ACC_ROWS, HID)[:N_NODES]


def kernel(x_vals, x_cons, edge_index, params):
    hv = _encode(x_vals, *params["enc_v"])
    hc = _encode(x_cons, *params["enc_c"])
    row = edge_index[0].astype(jnp.int32)
    col = edge_index[1].astype(jnp.int32)
    zrows = jnp.zeros((TILE_ROWS, FHALF), jnp.float32)
    for layer in params["convs"]:
        msg_c = _segment_sum(hv, col, row, zrows)
        hc = _update(msg_c, hc, layer["Wv2c"], layer["Wcs"], layer["bc"])
        msg_v = _segment_sum(hc, row, col, zrows)
        hv = _update(msg_v, hv, layer["Wc2v"], layer["Wvs"], layer["bv"])
    return _pred(hv, params["pred"][0], params["out"])


# speedup vs baseline: 1.9355x; 1.1005x over previous
import functools

import jax
import jax.numpy as jnp
from jax import lax
from jax.experimental import pallas as pl
from jax.experimental.pallas import tpu as pltpu
from jax.experimental.pallas import tpu_sc as plsc

HID = 64
FHALF = HID // 2
N_NODES = 50000
N_EDGES = 800000
TILE_ROWS = 3128
ACC_ROWS = 16 * TILE_ROWS
E_PER_TILE = N_EDGES // 16
SUPER = 2000
CHUNK = 80
NSUP = E_PER_TILE // SUPER
NCH = SUPER // CHUNK


def _encode_body(x_ref, w1_ref, b1_ref, w2_ref, b2_ref, o_ref):
    h = jnp.dot(x_ref[...], w1_ref[...], preferred_element_type=jnp.float32)
    h = jnp.maximum(h + b1_ref[...], 0.0)
    h = jnp.dot(h, w2_ref[...], preferred_element_type=jnp.float32)
    o_ref[...] = jnp.maximum(h + b2_ref[...], 0.0)


def _encode(x, p1, p2):
    n, din = x.shape
    blk = 2000
    return pl.pallas_call(
        _encode_body,
        grid=(n // blk,),
        in_specs=[
            pl.BlockSpec((blk, din), lambda i: (i, 0)),
            pl.BlockSpec((din, HID), lambda i: (0, 0)),
            pl.BlockSpec((1, HID), lambda i: (0, 0)),
            pl.BlockSpec((HID, HID), lambda i: (0, 0)),
            pl.BlockSpec((1, HID), lambda i: (0, 0)),
        ],
        out_specs=pl.BlockSpec((blk, HID), lambda i: (i, 0)),
        out_shape=jax.ShapeDtypeStruct((n, HID), jnp.float32),
    )(x, p1["W"], p1["b"].reshape(1, HID), p2["W"], p2["b"].reshape(1, HID))


def _update_body(m_ref, h_ref, wm_ref, wh_ref, b_ref, o_ref):
    z = (jnp.dot(m_ref[...], wm_ref[...], preferred_element_type=jnp.float32)
         + jnp.dot(h_ref[...], wh_ref[...], preferred_element_type=jnp.float32)
         + b_ref[...])
    mu = jnp.mean(z, axis=-1, keepdims=True)
    zc = z - mu
    var = jnp.mean(zc * zc, axis=-1, keepdims=True)
    o_ref[...] = jnp.maximum(zc * lax.rsqrt(var + 1e-5), 0.0)


def _update(msg, h, wm, wh, b):
    n = h.shape[0]
    blk = 2000
    return pl.pallas_call(
        _update_body,
        grid=(n // blk,),
        in_specs=[
            pl.BlockSpec((blk, HID), lambda i: (i, 0)),
            pl.BlockSpec((blk, HID), lambda i: (i, 0)),
            pl.BlockSpec((HID, HID), lambda i: (0, 0)),
            pl.BlockSpec((HID, HID), lambda i: (0, 0)),
            pl.BlockSpec((1, HID), lambda i: (0, 0)),
        ],
        out_specs=pl.BlockSpec((blk, HID), lambda i: (i, 0)),
        out_shape=jax.ShapeDtypeStruct((n, HID), jnp.float32),
    )(msg, h, wm, wh, b.reshape(1, HID))


def _pred_body(h_ref, wp_ref, bp_ref, wo_ref, bo_ref, o_ref):
    h = jnp.dot(h_ref[...], wp_ref[...], preferred_element_type=jnp.float32)
    h = jnp.maximum(h + bp_ref[...], 0.0)
    o_ref[...] = jnp.sum(h * wo_ref[...], axis=1) + bo_ref[0, 0]


def _pred(h, pred_p, out_p):
    n = h.shape[0]
    blk = 2048
    grid = (n + blk - 1) // blk
    out = pl.pallas_call(
        _pred_body,
        grid=(grid,),
        in_specs=[
            pl.BlockSpec((blk, HID), lambda i: (i, 0)),
            pl.BlockSpec((HID, HID), lambda i: (0, 0)),
            pl.BlockSpec((1, HID), lambda i: (0, 0)),
            pl.BlockSpec((1, HID), lambda i: (0, 0)),
            pl.BlockSpec((1, 1), lambda i: (0, 0)),
        ],
        out_specs=pl.BlockSpec((blk,), lambda i: (i,)),
        out_shape=jax.ShapeDtypeStruct((grid * blk,), jnp.float32),
    )(h, pred_p["W"], pred_p["b"].reshape(1, HID),
      out_p["W"].reshape(1, HID), out_p["b"].reshape(1, 1))
    return out[:n]


RING = 6
LEAD = 3
NSLOT = 3
TOT = NSUP * (SUPER // CHUNK)


def _segsum_body(table2, gidx, sidx, zrows, out, acc, gsb, ssb, gbufs, dbufs,
                 rows, gsem, ssem, tsem):
    c = lax.axis_index("c")
    s = lax.axis_index("s")
    tile_base = s * TILE_ROWS
    pltpu.sync_copy(zrows, acc.at[pl.ds(tile_base, TILE_ROWS)])
    plsc.subcore_barrier()

    ebase = s * E_PER_TILE

    def stage_cps(j):
        sl = lax.rem(j, NSLOT)
        sb = ebase + j * SUPER
        return (pltpu.make_async_copy(gidx.at[pl.ds(sb, SUPER)], gsb.at[sl],
                                      tsem.at[0]),
                pltpu.make_async_copy(sidx.at[pl.ds(sb, SUPER)], ssb.at[sl],
                                      tsem.at[1]))

    for cp in stage_cps(0):
        cp.start()

    def gather_cp(m):
        return pltpu.make_async_copy(table2.at[gbufs.at[m]], rows.at[m],
                                     gsem.at[m])

    def scatter_cp(m):
        return pltpu.make_async_copy(rows.at[m], acc.at[dbufs.at[m]],
                                     ssem.at[m])

    def ch_body(pp, carry):
        for b in range(RING):
            p = pp * RING + b
            sq = p - 2 * LEAD
            cq = p - LEAD
            ms = b
            mc = (b + RING - LEAD % RING) % RING
            mq = (b + RING - (2 * LEAD) % RING) % RING

            @pl.when((sq >= 0) & (sq < TOT))
            def _():
                scatter_cp(mq).wait()

            @pl.when(p < TOT)
            def _():
                j = lax.div(p, NCH)
                r = lax.rem(p, NCH)

                @pl.when(r == 0)
                def _():
                    for cp in stage_cps(j):
                        cp.wait()

                    @pl.when(j + 1 < NSUP)
                    def _():
                        for cp in stage_cps(j + 1):
                            cp.start()

                sl = lax.rem(j, NSLOT)
                off = r * CHUNK
                for t in range(CHUNK // 16):
                    g = gsb[sl, pl.ds(off + t * 16, 16)]
                    gbufs[ms, pl.ds(t * 16, 16)] = g + g + c
                gather_cp(ms).start()

            @pl.when((cq >= 0) & (cq < TOT))
            def _():
                j = lax.div(cq, NCH)
                sl = lax.rem(j, NSLOT)
                off = lax.rem(cq, NCH) * CHUNK
                gather_cp(mc).wait()
                for t in range(CHUNK // 16):
                    dbufs[mc, pl.ds(t * 16, 16)] = ssb[sl,
                                                       pl.ds(off + t * 16, 16)]
                scatter_cp(mc).start(add=True)
        return carry

    lax.fori_loop(0, (TOT + 2 * LEAD + RING - 1) // RING, ch_body, 0)
    plsc.subcore_barrier()
    pltpu.sync_copy(acc.at[pl.ds(tile_base, TILE_ROWS)],
                    out.at[pl.ds(tile_base, TILE_ROWS), c])


@functools.cache
def _segsum_call():
    return pl.kernel(
        _segsum_body,
        out_type=jax.ShapeDtypeStruct((ACC_ROWS, 2, FHALF), jnp.float32),
        mesh=plsc.VectorSubcoreMesh(core_axis_name="c", subcore_axis_name="s",
                                    num_cores=2, num_subcores=16),
        scratch_types=[
            pltpu.VMEM_SHARED((ACC_ROWS, FHALF), jnp.float32),
            pltpu.VMEM((NSLOT, SUPER), jnp.int32),
            pltpu.VMEM((NSLOT, SUPER), jnp.int32),
            pltpu.VMEM((RING, CHUNK), jnp.int32),
            pltpu.VMEM((RING, CHUNK), jnp.int32),
            pltpu.VMEM((RING, CHUNK, FHALF), jnp.float32),
            pltpu.SemaphoreType.DMA((RING,)),
            pltpu.SemaphoreType.DMA((RING,)),
            pltpu.SemaphoreType.DMA((2,)),
        ],
        compiler_params=pltpu.CompilerParams(use_tc_tiling_on_sc=False),
    )


def _segment_sum(table, g_idx, s_idx, zrows):
    table2 = table.reshape(2 * N_NODES, FHALF)
    out = _segsum_call()(table2, g_idx, s_idx, zrows)
    return out.reshape(ACC_ROWS, HID)[:N_NODES]


def kernel(x_vals, x_cons, edge_index, params):
    hv = _encode(x_vals, *params["enc_v"])
    hc = _encode(x_cons, *params["enc_c"])
    row = edge_index[0].astype(jnp.int32)
    col = edge_index[1].astype(jnp.int32)
    zrows = jnp.zeros((TILE_ROWS, FHALF), jnp.float32)
    for layer in params["convs"]:
        msg_c = _segment_sum(hv, col, row, zrows)
        hc = _update(msg_c, hc, layer["Wv2c"], layer["Wcs"], layer["bc"])
        msg_v = _segment_sum(hc, row, col, zrows)
        hv = _update(msg_v, hv, layer["Wc2v"], layer["Wvs"], layer["bv"])
    return _pred(hv, params["pred"][0], params["out"])


# speedup vs baseline: 3.1743x; 1.6400x over previous
import functools

import jax
import jax.numpy as jnp
from jax import lax
from jax.experimental import pallas as pl
from jax.experimental.pallas import tpu as pltpu
from jax.experimental.pallas import tpu_sc as plsc

HID = 64
FHALF = HID // 2
N_NODES = 50000
N_EDGES = 800000
TILE_ROWS = 3128
ACC_ROWS = 16 * TILE_ROWS
E_PER_TILE = N_EDGES // 16
SUPER = 2000
CHUNK = 80
NSUP = E_PER_TILE // SUPER
NCH = SUPER // CHUNK


def _encode_body(x_ref, w1_ref, b1_ref, w2_ref, b2_ref, o_ref):
    h = jnp.dot(x_ref[...], w1_ref[...], preferred_element_type=jnp.float32)
    h = jnp.maximum(h + b1_ref[...], 0.0)
    h = jnp.dot(h, w2_ref[...], preferred_element_type=jnp.float32)
    o_ref[...] = jnp.maximum(h + b2_ref[...], 0.0)


def _encode(x, p1, p2):
    n, din = x.shape
    blk = 2000
    return pl.pallas_call(
        _encode_body,
        grid=(n // blk,),
        in_specs=[
            pl.BlockSpec((blk, din), lambda i: (i, 0)),
            pl.BlockSpec((din, HID), lambda i: (0, 0)),
            pl.BlockSpec((1, HID), lambda i: (0, 0)),
            pl.BlockSpec((HID, HID), lambda i: (0, 0)),
            pl.BlockSpec((1, HID), lambda i: (0, 0)),
        ],
        out_specs=pl.BlockSpec((blk, HID), lambda i: (i, 0)),
        out_shape=jax.ShapeDtypeStruct((n, HID), jnp.float32),
    )(x, p1["W"], p1["b"].reshape(1, HID), p2["W"], p2["b"].reshape(1, HID))


def _update_body(m_ref, h_ref, wm_ref, wh_ref, b_ref, o_ref):
    z = (jnp.dot(m_ref[...], wm_ref[...], preferred_element_type=jnp.float32)
         + jnp.dot(h_ref[...], wh_ref[...], preferred_element_type=jnp.float32)
         + b_ref[...])
    mu = jnp.mean(z, axis=-1, keepdims=True)
    zc = z - mu
    var = jnp.mean(zc * zc, axis=-1, keepdims=True)
    o_ref[...] = jnp.maximum(zc * lax.rsqrt(var + 1e-5), 0.0)


def _update(msg, h, wm, wh, b):
    n = h.shape[0]
    blk = 2000
    return pl.pallas_call(
        _update_body,
        grid=(n // blk,),
        in_specs=[
            pl.BlockSpec((blk, HID), lambda i: (i, 0)),
            pl.BlockSpec((blk, HID), lambda i: (i, 0)),
            pl.BlockSpec((HID, HID), lambda i: (0, 0)),
            pl.BlockSpec((HID, HID), lambda i: (0, 0)),
            pl.BlockSpec((1, HID), lambda i: (0, 0)),
        ],
        out_specs=pl.BlockSpec((blk, HID), lambda i: (i, 0)),
        out_shape=jax.ShapeDtypeStruct((n, HID), jnp.float32),
    )(msg, h, wm, wh, b.reshape(1, HID))


def _pred_body(h_ref, wp_ref, bp_ref, wo_ref, bo_ref, o_ref):
    h = jnp.dot(h_ref[...], wp_ref[...], preferred_element_type=jnp.float32)
    h = jnp.maximum(h + bp_ref[...], 0.0)
    o_ref[...] = jnp.sum(h * wo_ref[...], axis=1) + bo_ref[0, 0]


def _pred(h, pred_p, out_p):
    n = h.shape[0]
    blk = 2048
    grid = (n + blk - 1) // blk
    out = pl.pallas_call(
        _pred_body,
        grid=(grid,),
        in_specs=[
            pl.BlockSpec((blk, HID), lambda i: (i, 0)),
            pl.BlockSpec((HID, HID), lambda i: (0, 0)),
            pl.BlockSpec((1, HID), lambda i: (0, 0)),
            pl.BlockSpec((1, HID), lambda i: (0, 0)),
            pl.BlockSpec((1, 1), lambda i: (0, 0)),
        ],
        out_specs=pl.BlockSpec((blk,), lambda i: (i,)),
        out_shape=jax.ShapeDtypeStruct((grid * blk,), jnp.float32),
    )(h, pred_p["W"], pred_p["b"].reshape(1, HID),
      out_p["W"].reshape(1, HID), out_p["b"].reshape(1, 1))
    return out[:n]


RING = 6
LEAD = 3
NSLOT = 3
TOT = NSUP * (SUPER // CHUNK)


def _segsum_body(table2, gidx, sidx, zrows, out, acc, gsb, ssb, gbufs, dbufs,
                 rows, gsem, ssem, tsem):
    c = lax.axis_index("c")
    s = lax.axis_index("s")
    tile_base = s * TILE_ROWS
    pltpu.sync_copy(zrows, acc.at[pl.ds(tile_base, TILE_ROWS)])
    plsc.subcore_barrier()

    ebase = s * E_PER_TILE

    def stage_cps(j):
        sl = lax.rem(j, NSLOT)
        sb = ebase + j * SUPER
        return (pltpu.make_async_copy(gidx.at[pl.ds(sb, SUPER)], gsb.at[sl],
                                      tsem.at[0]),
                pltpu.make_async_copy(sidx.at[pl.ds(sb, SUPER)], ssb.at[sl],
                                      tsem.at[1]))

    for cp in stage_cps(0):
        cp.start()

    def gather_cp(m):
        return pltpu.make_async_copy(table2.at[gbufs.at[m]], rows.at[m],
                                     gsem.at[m])

    def scatter_cp(m):
        return pltpu.make_async_copy(rows.at[m], acc.at[dbufs.at[m]],
                                     ssem.at[m])

    def ch_body(pp, carry):
        for b in range(RING):
            p = pp * RING + b
            sq = p - 2 * LEAD
            cq = p - LEAD
            ms = b
            mc = (b + RING - LEAD % RING) % RING
            mq = (b + RING - (2 * LEAD) % RING) % RING

            @pl.when((sq >= 0) & (sq < TOT))
            def _():
                scatter_cp(mq).wait()

            @pl.when(p < TOT)
            def _():
                j = lax.div(p, NCH)
                r = lax.rem(p, NCH)

                @pl.when(r == 0)
                def _():
                    for cp in stage_cps(j):
                        cp.wait()

                    @pl.when(j + 1 < NSUP)
                    def _():
                        for cp in stage_cps(j + 1):
                            cp.start()

                sl = lax.rem(j, NSLOT)
                off = r * CHUNK
                for t in range(CHUNK // 16):
                    g = gsb[sl, pl.ds(off + t * 16, 16)]
                    gbufs[ms, pl.ds(t * 16, 16)] = g + g + c
                gather_cp(ms).start()

            @pl.when((cq >= 0) & (cq < TOT))
            def _():
                j = lax.div(cq, NCH)
                sl = lax.rem(j, NSLOT)
                off = lax.rem(cq, NCH) * CHUNK
                gather_cp(mc).wait()
                for t in range(CHUNK // 16):
                    dbufs[mc, pl.ds(t * 16, 16)] = ssb[sl,
                                                       pl.ds(off + t * 16, 16)]
                scatter_cp(mc).start(add=True)
        return carry

    lax.fori_loop(0, (TOT + 2 * LEAD + RING - 1) // RING, ch_body, 0)
    plsc.subcore_barrier()
    pltpu.sync_copy(acc.at[pl.ds(tile_base, TILE_ROWS)],
                    out.at[pl.ds(tile_base, TILE_ROWS),
                           pl.ds(c * FHALF, FHALF)])


@functools.cache
def _segsum_call():
    return pl.kernel(
        _segsum_body,
        out_type=jax.ShapeDtypeStruct((ACC_ROWS, HID), jnp.float32),
        mesh=plsc.VectorSubcoreMesh(core_axis_name="c", subcore_axis_name="s",
                                    num_cores=2, num_subcores=16),
        scratch_types=[
            pltpu.VMEM_SHARED((ACC_ROWS, FHALF), jnp.float32),
            pltpu.VMEM((NSLOT, SUPER), jnp.int32),
            pltpu.VMEM((NSLOT, SUPER), jnp.int32),
            pltpu.VMEM((RING, CHUNK), jnp.int32),
            pltpu.VMEM((RING, CHUNK), jnp.int32),
            pltpu.VMEM((RING, CHUNK, FHALF), jnp.float32),
            pltpu.SemaphoreType.DMA((RING,)),
            pltpu.SemaphoreType.DMA((RING,)),
            pltpu.SemaphoreType.DMA((2,)),
        ],
        compiler_params=pltpu.CompilerParams(use_tc_tiling_on_sc=False),
    )


def _segment_sum(table, g_idx, s_idx, zrows):
    table2 = table.reshape(2 * N_NODES, FHALF)
    return _segsum_call()(table2, g_idx, s_idx, zrows)


def kernel(x_vals, x_cons, edge_index, params):
    hv = _encode(x_vals, *params["enc_v"])
    hc = _encode(x_cons, *params["enc_c"])
    row = edge_index[0].astype(jnp.int32)
    col = edge_index[1].astype(jnp.int32)
    zrows = jnp.zeros((TILE_ROWS, FHALF), jnp.float32)
    for layer in params["convs"]:
        msg_c = _segment_sum(hv, col, row, zrows)
        hc = _update(msg_c, hc, layer["Wv2c"], layer["Wcs"], layer["bc"])
        msg_v = _segment_sum(hc, row, col, zrows)
        hv = _update(msg_v, hv, layer["Wc2v"], layer["Wvs"], layer["bv"])
    return _pred(hv, params["pred"][0], params["out"])


# speedup vs baseline: 3.3166x; 1.0448x over previous
import functools

import jax
import jax.numpy as jnp
from jax import lax
from jax.experimental import pallas as pl
from jax.experimental.pallas import tpu as pltpu
from jax.experimental.pallas import tpu_sc as plsc

HID = 64
FHALF = HID // 2
N_NODES = 50000
N_EDGES = 800000
TILE_ROWS = 3128
ACC_ROWS = 16 * TILE_ROWS
E_PER_TILE = N_EDGES // 16
SUPER = 2000
CHUNK = 80
NSUP = E_PER_TILE // SUPER
NCH = SUPER // CHUNK


def _encode_body(x_ref, w1_ref, b1_ref, w2_ref, b2_ref, o_ref):
    h = jnp.dot(x_ref[...], w1_ref[...], preferred_element_type=jnp.float32)
    h = jnp.maximum(h + b1_ref[...], 0.0)
    h = jnp.dot(h, w2_ref[...], preferred_element_type=jnp.float32)
    o_ref[...] = jnp.maximum(h + b2_ref[...], 0.0)


def _encode(x, p1, p2):
    n, din = x.shape
    blk = 10000
    return pl.pallas_call(
        _encode_body,
        grid=(n // blk,),
        in_specs=[
            pl.BlockSpec((blk, din), lambda i: (i, 0)),
            pl.BlockSpec((din, HID), lambda i: (0, 0)),
            pl.BlockSpec((1, HID), lambda i: (0, 0)),
            pl.BlockSpec((HID, HID), lambda i: (0, 0)),
            pl.BlockSpec((1, HID), lambda i: (0, 0)),
        ],
        out_specs=pl.BlockSpec((blk, HID), lambda i: (i, 0)),
        out_shape=jax.ShapeDtypeStruct((n, HID), jnp.float32),
    )(x, p1["W"], p1["b"].reshape(1, HID), p2["W"], p2["b"].reshape(1, HID))


def _update_body(m_ref, h_ref, wm_ref, wh_ref, b_ref, o_ref):
    z = (jnp.dot(m_ref[...], wm_ref[...], preferred_element_type=jnp.float32)
         + jnp.dot(h_ref[...], wh_ref[...], preferred_element_type=jnp.float32)
         + b_ref[...])
    mu = jnp.mean(z, axis=-1, keepdims=True)
    zc = z - mu
    var = jnp.mean(zc * zc, axis=-1, keepdims=True)
    o_ref[...] = jnp.maximum(zc * lax.rsqrt(var + 1e-5), 0.0)


def _update(msg, h, wm, wh, b):
    n = h.shape[0]
    blk = 10000
    return pl.pallas_call(
        _update_body,
        grid=(n // blk,),
        in_specs=[
            pl.BlockSpec((blk, HID), lambda i: (i, 0)),
            pl.BlockSpec((blk, HID), lambda i: (i, 0)),
            pl.BlockSpec((HID, HID), lambda i: (0, 0)),
            pl.BlockSpec((HID, HID), lambda i: (0, 0)),
            pl.BlockSpec((1, HID), lambda i: (0, 0)),
        ],
        out_specs=pl.BlockSpec((blk, HID), lambda i: (i, 0)),
        out_shape=jax.ShapeDtypeStruct((n, HID), jnp.float32),
    )(msg, h, wm, wh, b.reshape(1, HID))


def _update_pred_body(m_ref, h_ref, wm_ref, wh_ref, b_ref, wp_ref, bp_ref,
                      wo_ref, bo_ref, o_ref):
    z = (jnp.dot(m_ref[...], wm_ref[...], preferred_element_type=jnp.float32)
         + jnp.dot(h_ref[...], wh_ref[...], preferred_element_type=jnp.float32)
         + b_ref[...])
    mu = jnp.mean(z, axis=-1, keepdims=True)
    zc = z - mu
    var = jnp.mean(zc * zc, axis=-1, keepdims=True)
    hv = jnp.maximum(zc * lax.rsqrt(var + 1e-5), 0.0)
    hp = jnp.dot(hv, wp_ref[...], preferred_element_type=jnp.float32)
    hp = jnp.maximum(hp + bp_ref[...], 0.0)
    o_ref[...] = jnp.sum(hp * wo_ref[...], axis=1) + bo_ref[0, 0]


def _update_pred(msg, h, wm, wh, b, pred_p, out_p):
    n = h.shape[0]
    blk = 2048
    grid = (n + blk - 1) // blk
    out = pl.pallas_call(
        _update_pred_body,
        grid=(grid,),
        in_specs=[
            pl.BlockSpec((blk, HID), lambda i: (i, 0)),
            pl.BlockSpec((blk, HID), lambda i: (i, 0)),
            pl.BlockSpec((HID, HID), lambda i: (0, 0)),
            pl.BlockSpec((HID, HID), lambda i: (0, 0)),
            pl.BlockSpec((1, HID), lambda i: (0, 0)),
            pl.BlockSpec((HID, HID), lambda i: (0, 0)),
            pl.BlockSpec((1, HID), lambda i: (0, 0)),
            pl.BlockSpec((1, HID), lambda i: (0, 0)),
            pl.BlockSpec((1, 1), lambda i: (0, 0)),
        ],
        out_specs=pl.BlockSpec((blk,), lambda i: (i,)),
        out_shape=jax.ShapeDtypeStruct((grid * blk,), jnp.float32),
    )(msg, h, wm, wh, b.reshape(1, HID), pred_p["W"],
      pred_p["b"].reshape(1, HID), out_p["W"].reshape(1, HID),
      out_p["b"].reshape(1, 1))
    return out[:n]


RING = 6
LEAD = 3
NSLOT = 3
TOT = NSUP * (SUPER // CHUNK)


def _segsum_body(table2, gidx, sidx, zrows, out, acc, gsb, ssb, gbufs, dbufs,
                 rows, gsem, ssem, tsem):
    c = lax.axis_index("c")
    s = lax.axis_index("s")
    tile_base = s * TILE_ROWS
    pltpu.sync_copy(zrows, acc.at[pl.ds(tile_base, TILE_ROWS)])
    plsc.subcore_barrier()

    ebase = s * E_PER_TILE

    def stage_cps(j):
        sl = lax.rem(j, NSLOT)
        sb = ebase + j * SUPER
        return (pltpu.make_async_copy(gidx.at[pl.ds(sb, SUPER)], gsb.at[sl],
                                      tsem.at[0]),
                pltpu.make_async_copy(sidx.at[pl.ds(sb, SUPER)], ssb.at[sl],
                                      tsem.at[1]))

    for cp in stage_cps(0):
        cp.start()

    def gather_cp(m):
        return pltpu.make_async_copy(table2.at[gbufs.at[m]], rows.at[m],
                                     gsem.at[m])

    def scatter_cp(m):
        return pltpu.make_async_copy(rows.at[m], acc.at[dbufs.at[m]],
                                     ssem.at[m])

    def ch_body(pp, carry):
        for b in range(RING):
            p = pp * RING + b
            sq = p - 2 * LEAD
            cq = p - LEAD
            ms = b
            mc = (b + RING - LEAD % RING) % RING
            mq = (b + RING - (2 * LEAD) % RING) % RING

            @pl.when((sq >= 0) & (sq < TOT))
            def _():
                scatter_cp(mq).wait()

            @pl.when(p < TOT)
            def _():
                j = lax.div(p, NCH)
                r = lax.rem(p, NCH)

                @pl.when(r == 0)
                def _():
                    for cp in stage_cps(j):
                        cp.wait()

                    @pl.when(j + 1 < NSUP)
                    def _():
                        for cp in stage_cps(j + 1):
                            cp.start()

                sl = lax.rem(j, NSLOT)
                off = r * CHUNK
                for t in range(CHUNK // 16):
                    g = gsb[sl, pl.ds(off + t * 16, 16)]
                    gbufs[ms, pl.ds(t * 16, 16)] = g + g + c
                gather_cp(ms).start()

            @pl.when((cq >= 0) & (cq < TOT))
            def _():
                j = lax.div(cq, NCH)
                sl = lax.rem(j, NSLOT)
                off = lax.rem(cq, NCH) * CHUNK
                gather_cp(mc).wait()
                for t in range(CHUNK // 16):
                    dbufs[mc, pl.ds(t * 16, 16)] = ssb[sl,
                                                       pl.ds(off + t * 16, 16)]
                scatter_cp(mc).start(add=True)
        return carry

    lax.fori_loop(0, (TOT + 2 * LEAD + RING - 1) // RING, ch_body, 0)
    plsc.subcore_barrier()
    pltpu.sync_copy(acc.at[pl.ds(tile_base, TILE_ROWS)],
                    out.at[pl.ds(tile_base, TILE_ROWS),
                           pl.ds(c * FHALF, FHALF)])


@functools.cache
def _segsum_call():
    return pl.kernel(
        _segsum_body,
        out_type=jax.ShapeDtypeStruct((ACC_ROWS, HID), jnp.float32),
        mesh=plsc.VectorSubcoreMesh(core_axis_name="c", subcore_axis_name="s",
                                    num_cores=2, num_subcores=16),
        scratch_types=[
            pltpu.VMEM_SHARED((ACC_ROWS, FHALF), jnp.float32),
            pltpu.VMEM((NSLOT, SUPER), jnp.int32),
            pltpu.VMEM((NSLOT, SUPER), jnp.int32),
            pltpu.VMEM((RING, CHUNK), jnp.int32),
            pltpu.VMEM((RING, CHUNK), jnp.int32),
            pltpu.VMEM((RING, CHUNK, FHALF), jnp.float32),
            pltpu.SemaphoreType.DMA((RING,)),
            pltpu.SemaphoreType.DMA((RING,)),
            pltpu.SemaphoreType.DMA((2,)),
        ],
        compiler_params=pltpu.CompilerParams(use_tc_tiling_on_sc=False),
    )


def _segment_sum(table, g_idx, s_idx, zrows):
    table2 = table.reshape(2 * N_NODES, FHALF)
    return _segsum_call()(table2, g_idx, s_idx, zrows)


def kernel(x_vals, x_cons, edge_index, params):
    hv = _encode(x_vals, *params["enc_v"])
    hc = _encode(x_cons, *params["enc_c"])
    row = edge_index[0].astype(jnp.int32)
    col = edge_index[1].astype(jnp.int32)
    zrows = jnp.zeros((TILE_ROWS, FHALF), jnp.float32)
    for li, layer in enumerate(params["convs"]):
        msg_c = _segment_sum(hv, col, row, zrows)
        hc = _update(msg_c, hc, layer["Wv2c"], layer["Wcs"], layer["bc"])
        msg_v = _segment_sum(hc, row, col, zrows)
        if li + 1 < len(params["convs"]):
            hv = _update(msg_v, hv, layer["Wc2v"], layer["Wvs"], layer["bv"])
        else:
            return _update_pred(msg_v, hv, layer["Wc2v"], layer["Wvs"],
                                layer["bv"], params["pred"][0], params["out"])


# speedup vs baseline: 3.3882x; 1.0216x over previous
import functools

import jax
import jax.numpy as jnp
from jax import lax
from jax.experimental import pallas as pl
from jax.experimental.pallas import tpu as pltpu
from jax.experimental.pallas import tpu_sc as plsc

HID = 64
FHALF = HID // 2
N_NODES = 50000
N_EDGES = 800000
TILE_ROWS = 3128
ACC_ROWS = 16 * TILE_ROWS
E_PER_TILE = N_EDGES // 16
SUPER = 2000
CHUNK = 80
NSUP = E_PER_TILE // SUPER
NCH = SUPER // CHUNK


def _encode_body(x_ref, w1_ref, b1_ref, w2_ref, b2_ref, o_ref):
    h = jnp.dot(x_ref[...], w1_ref[...], preferred_element_type=jnp.float32)
    h = jnp.maximum(h + b1_ref[...], 0.0)
    h = jnp.dot(h, w2_ref[...], preferred_element_type=jnp.float32)
    o_ref[...] = jnp.maximum(h + b2_ref[...], 0.0)


def _encode(x, p1, p2):
    n, din = x.shape
    blk = 10000
    return pl.pallas_call(
        _encode_body,
        grid=(n // blk,),
        in_specs=[
            pl.BlockSpec((blk, din), lambda i: (i, 0)),
            pl.BlockSpec((din, HID), lambda i: (0, 0)),
            pl.BlockSpec((1, HID), lambda i: (0, 0)),
            pl.BlockSpec((HID, HID), lambda i: (0, 0)),
            pl.BlockSpec((1, HID), lambda i: (0, 0)),
        ],
        out_specs=pl.BlockSpec((blk, HID), lambda i: (i, 0)),
        out_shape=jax.ShapeDtypeStruct((n, HID), jnp.float32),
    )(x, p1["W"], p1["b"].reshape(1, HID), p2["W"], p2["b"].reshape(1, HID))


def _update_body(m_ref, h_ref, wm_ref, wh_ref, b_ref, o_ref):
    z = (jnp.dot(m_ref[...], wm_ref[...], preferred_element_type=jnp.float32)
         + jnp.dot(h_ref[...], wh_ref[...], preferred_element_type=jnp.float32)
         + b_ref[...])
    mu = jnp.mean(z, axis=-1, keepdims=True)
    zc = z - mu
    var = jnp.mean(zc * zc, axis=-1, keepdims=True)
    o_ref[...] = jnp.maximum(zc * lax.rsqrt(var + 1e-5), 0.0)


def _update(msg, h, wm, wh, b):
    n = h.shape[0]
    blk = 10000
    return pl.pallas_call(
        _update_body,
        grid=(n // blk,),
        in_specs=[
            pl.BlockSpec((blk, HID), lambda i: (i, 0)),
            pl.BlockSpec((blk, HID), lambda i: (i, 0)),
            pl.BlockSpec((HID, HID), lambda i: (0, 0)),
            pl.BlockSpec((HID, HID), lambda i: (0, 0)),
            pl.BlockSpec((1, HID), lambda i: (0, 0)),
        ],
        out_specs=pl.BlockSpec((blk, HID), lambda i: (i, 0)),
        out_shape=jax.ShapeDtypeStruct((n, HID), jnp.float32),
    )(msg, h, wm, wh, b.reshape(1, HID))


def _update_pred_body(m_ref, h_ref, wm_ref, wh_ref, b_ref, wp_ref, bp_ref,
                      wo_ref, bo_ref, o_ref):
    z = (jnp.dot(m_ref[...], wm_ref[...], preferred_element_type=jnp.float32)
         + jnp.dot(h_ref[...], wh_ref[...], preferred_element_type=jnp.float32)
         + b_ref[...])
    mu = jnp.mean(z, axis=-1, keepdims=True)
    zc = z - mu
    var = jnp.mean(zc * zc, axis=-1, keepdims=True)
    hv = jnp.maximum(zc * lax.rsqrt(var + 1e-5), 0.0)
    hp = jnp.dot(hv, wp_ref[...], preferred_element_type=jnp.float32)
    hp = jnp.maximum(hp + bp_ref[...], 0.0)
    o_ref[...] = jnp.sum(hp * wo_ref[...], axis=1) + bo_ref[0, 0]


def _update_pred(msg, h, wm, wh, b, pred_p, out_p):
    n = h.shape[0]
    blk = 2048
    grid = (n + blk - 1) // blk
    out = pl.pallas_call(
        _update_pred_body,
        grid=(grid,),
        in_specs=[
            pl.BlockSpec((blk, HID), lambda i: (i, 0)),
            pl.BlockSpec((blk, HID), lambda i: (i, 0)),
            pl.BlockSpec((HID, HID), lambda i: (0, 0)),
            pl.BlockSpec((HID, HID), lambda i: (0, 0)),
            pl.BlockSpec((1, HID), lambda i: (0, 0)),
            pl.BlockSpec((HID, HID), lambda i: (0, 0)),
            pl.BlockSpec((1, HID), lambda i: (0, 0)),
            pl.BlockSpec((1, HID), lambda i: (0, 0)),
            pl.BlockSpec((1, 1), lambda i: (0, 0)),
        ],
        out_specs=pl.BlockSpec((blk,), lambda i: (i,)),
        out_shape=jax.ShapeDtypeStruct((grid * blk,), jnp.float32),
    )(msg, h, wm, wh, b.reshape(1, HID), pred_p["W"],
      pred_p["b"].reshape(1, HID), out_p["W"].reshape(1, HID),
      out_p["b"].reshape(1, 1))
    return out[:n]


RING = 6
LEAD = 3
NSLOT = 3
TOT = NSUP * (SUPER // CHUNK)


def _segsum_body(dg, table2, ei, zrows, out, acc, gsb, ssb, gbufs, dbufs,
                 rows, gsem, ssem, tsem):
    c = lax.axis_index("c")
    s = lax.axis_index("s")
    tile_base = s * TILE_ROWS
    pltpu.sync_copy(zrows, acc.at[pl.ds(tile_base, TILE_ROWS)])
    plsc.subcore_barrier()

    ebase = s * E_PER_TILE

    def stage_cps(j):
        sl = lax.rem(j, NSLOT)
        sb = ebase + j * SUPER
        return (pltpu.make_async_copy(ei.at[dg, pl.ds(sb, SUPER)], gsb.at[sl],
                                      tsem.at[0]),
                pltpu.make_async_copy(ei.at[1 - dg, pl.ds(sb, SUPER)],
                                      ssb.at[sl], tsem.at[1]))

    for cp in stage_cps(0):
        cp.start()

    def gather_cp(m):
        return pltpu.make_async_copy(table2.at[gbufs.at[m]], rows.at[m],
                                     gsem.at[m])

    def scatter_cp(m):
        return pltpu.make_async_copy(rows.at[m], acc.at[dbufs.at[m]],
                                     ssem.at[m])

    def ch_body(pp, carry):
        for b in range(RING):
            p = pp * RING + b
            sq = p - 2 * LEAD
            cq = p - LEAD
            ms = b
            mc = (b + RING - LEAD % RING) % RING
            mq = (b + RING - (2 * LEAD) % RING) % RING

            @pl.when((sq >= 0) & (sq < TOT))
            def _():
                scatter_cp(mq).wait()

            @pl.when(p < TOT)
            def _():
                j = lax.div(p, NCH)
                r = lax.rem(p, NCH)

                @pl.when(r == 0)
                def _():
                    for cp in stage_cps(j):
                        cp.wait()

                    @pl.when(j + 1 < NSUP)
                    def _():
                        for cp in stage_cps(j + 1):
                            cp.start()

                sl = lax.rem(j, NSLOT)
                off = r * CHUNK
                for t in range(CHUNK // 16):
                    g = gsb[sl, pl.ds(off + t * 16, 16)]
                    gbufs[ms, pl.ds(t * 16, 16)] = g + g + c
                gather_cp(ms).start()

            @pl.when((cq >= 0) & (cq < TOT))
            def _():
                j = lax.div(cq, NCH)
                sl = lax.rem(j, NSLOT)
                off = lax.rem(cq, NCH) * CHUNK
                gather_cp(mc).wait()
                for t in range(CHUNK // 16):
                    dbufs[mc, pl.ds(t * 16, 16)] = ssb[sl,
                                                       pl.ds(off + t * 16, 16)]
                scatter_cp(mc).start(add=True)
        return carry

    lax.fori_loop(0, (TOT + 2 * LEAD + RING - 1) // RING, ch_body, 0)
    plsc.subcore_barrier()
    pltpu.sync_copy(acc.at[pl.ds(tile_base, TILE_ROWS)],
                    out.at[pl.ds(tile_base, TILE_ROWS),
                           pl.ds(c * FHALF, FHALF)])


@functools.cache
def _segsum_call(dg):
    return pl.kernel(
        functools.partial(_segsum_body, dg),
        out_type=jax.ShapeDtypeStruct((ACC_ROWS, HID), jnp.float32),
        mesh=plsc.VectorSubcoreMesh(core_axis_name="c", subcore_axis_name="s",
                                    num_cores=2, num_subcores=16),
        scratch_types=[
            pltpu.VMEM_SHARED((ACC_ROWS, FHALF), jnp.float32),
            pltpu.VMEM((NSLOT, SUPER), jnp.int32),
            pltpu.VMEM((NSLOT, SUPER), jnp.int32),
            pltpu.VMEM((RING, CHUNK), jnp.int32),
            pltpu.VMEM((RING, CHUNK), jnp.int32),
            pltpu.VMEM((RING, CHUNK, FHALF), jnp.float32),
            pltpu.SemaphoreType.DMA((RING,)),
            pltpu.SemaphoreType.DMA((RING,)),
            pltpu.SemaphoreType.DMA((2,)),
        ],
        compiler_params=pltpu.CompilerParams(use_tc_tiling_on_sc=False),
    )


def _segment_sum(table, ei, dg, zrows):
    table2 = table.reshape(2 * N_NODES, FHALF)
    return _segsum_call(dg)(table2, ei, zrows)


def kernel(x_vals, x_cons, edge_index, params):
    hv = _encode(x_vals, *params["enc_v"])
    hc = _encode(x_cons, *params["enc_c"])
    ei = edge_index.astype(jnp.int32)
    zrows = jnp.zeros((TILE_ROWS, FHALF), jnp.float32)
    for li, layer in enumerate(params["convs"]):
        msg_c = _segment_sum(hv, ei, 1, zrows)
        hc = _update(msg_c, hc, layer["Wv2c"], layer["Wcs"], layer["bc"])
        msg_v = _segment_sum(hc, ei, 0, zrows)
        if li + 1 < len(params["convs"]):
            hv = _update(msg_v, hv, layer["Wc2v"], layer["Wvs"], layer["bv"])
        else:
            return _update_pred(msg_v, hv, layer["Wc2v"], layer["Wvs"],
                                layer["bv"], params["pred"][0], params["out"])
